# baseline (speedup 1.0000x reference)
import functools

import jax
import jax.numpy as jnp
from jax import lax
from jax.experimental import pallas as pl
from jax.experimental.pallas import tpu as pltpu
from jax.experimental.pallas import tpu_sc as plsc

NNODES = 10000
D = 128
NC = 2
NS = 16
NTILES = NC * NS
CHD = 128
NPAD = 10112
RPT = NPAD // NS
RPB = 2528
NBLK = NPAD // RPB


def _sc_mesh():
    return plsc.VectorSubcoreMesh(core_axis_name="c", subcore_axis_name="s")


def _row_chunks(total, step):
    out = []
    q0 = 0
    while q0 < total:
        out.append((q0, min(step, total - q0)))
        q0 += step
    return out


def _chunk_bounds(nch):
    return [8 * (nch * w // (NTILES * 8)) for w in range(NTILES + 1)]


@functools.partial(jax.jit, static_argnums=(1, 2, 3))
def _deg_call(ei3, nch8, max_n, rem):
    @functools.partial(
        pl.kernel,
        out_type=jax.ShapeDtypeStruct((NC * NPAD,), jnp.float32),
        mesh=_sc_mesh(),
        scratch_types=[
            pltpu.VMEM((max_n, 2, CHD), jnp.int32),
            pltpu.VMEM((CHD,), jnp.float32),
            pltpu.VMEM((RPT,), jnp.float32),
            pltpu.VMEM_SHARED((NPAD,), jnp.float32),
            pltpu.SemaphoreType.DMA,
        ],
    )
    def deg_kernel(ei_hbm, zrow_hbm, ones_hbm, out_hbm, eidx, ones_v,
                   stage_v, acc_sh, dsem):
        c = lax.axis_index("c")
        s = lax.axis_index("s")
        w = c * NS + s
        rw = 8 * ((nch8 * w) // (NTILES * 8))
        rw1 = 8 * ((nch8 * (w + 1)) // (NTILES * 8))
        n_w = rw1 - rw
        ld_ones = pltpu.make_async_copy(ones_hbm, ones_v, dsem)
        ld_idx = pltpu.make_async_copy(ei_hbm.at[pl.ds(rw, max_n)], eidx,
                                       dsem)
        ld_ones.start()
        ld_idx.start()
        pltpu.sync_copy(zrow_hbm, stage_v)
        pltpu.sync_copy(stage_v, acc_sh.at[pl.ds(s * RPT, RPT)])
        ld_ones.wait()
        ld_idx.wait()
        plsc.subcore_barrier()

        def body(j, carry):
            pltpu.async_copy(ones_v, acc_sh.at[eidx.at[j, 1]], dsem,
                             add=True)
            return carry

        lax.fori_loop(0, n_w, body, 0)

        def drain(j, carry):
            pltpu.make_async_copy(ones_v, acc_sh.at[eidx.at[0, 1]],
                                  dsem).wait()
            return carry

        lax.fori_loop(0, n_w, drain, 0)

        if rem:
            @pl.when(w < rem)
            def _tail():
                pltpu.sync_copy(ei_hbm.at[nch8 + w], eidx.at[0])
                pltpu.async_copy(ones_v, acc_sh.at[eidx.at[0, 1]], dsem,
                                 add=True)
                pltpu.make_async_copy(ones_v, acc_sh.at[eidx.at[0, 1]],
                                      dsem).wait()

        plsc.subcore_barrier()
        pltpu.sync_copy(acc_sh.at[pl.ds(s * RPT, RPT)], stage_v)
        pltpu.sync_copy(stage_v, out_hbm.at[pl.ds(c * NPAD + s * RPT, RPT)])

    zrow = jnp.zeros((RPT,), jnp.float32)
    ones = jnp.ones((CHD,), jnp.float32)
    return deg_kernel(ei3, zrow, ones)


@functools.partial(jax.jit, static_argnums=(2, 3))
def _agg_call(hp, ei3, nch8, rem):
    wb_chunks = _row_chunks(RPT, CHD)

    @functools.partial(
        pl.kernel,
        out_type=jax.ShapeDtypeStruct((NC * NPAD, D), jnp.float32),
        mesh=_sc_mesh(),
        scratch_types=[
            pltpu.VMEM((3, 2, CHD), jnp.int32),
            pltpu.VMEM((3, CHD, D), jnp.float32),
            pltpu.VMEM_SHARED((NPAD, D), jnp.float32),
        ] + [pltpu.SemaphoreType.DMA] * 9,
    )
    def agg_kernel(hp_hbm, ei_hbm, zrows_hbm, out_hbm,
                   eidx, rows, acc_sh,
                   g0, g1, g2, t0, t1, t2, d0, d1, d2):
        gs = (g0, g1, g2)
        ts = (t0, t1, t2)
        ds_ = (d0, d1, d2)
        c = lax.axis_index("c")
        s = lax.axis_index("s")
        w = c * NS + s
        r0 = s * RPT
        rw = 8 * ((nch8 * w) // (NTILES * 8))
        rw1 = 8 * ((nch8 * (w + 1)) // (NTILES * 8))
        n_w = rw1 - rw

        pltpu.sync_copy(zrows_hbm, rows.at[0])

        def _zinit(i, phase):
            q0, qn = wb_chunks[i]
            cp = pltpu.make_async_copy(rows.at[0, pl.ds(0, qn)],
                                       acc_sh.at[pl.ds(r0 + q0, qn), :],
                                       ds_[i % 3])
            cp.start() if phase == 0 else cp.wait()

        for i in range(len(wb_chunks)):
            _zinit(i, 0)
        for i in range(len(wb_chunks)):
            _zinit(i, 1)
        plsc.subcore_barrier()

        def eidx_cp(j, e):
            return pltpu.make_async_copy(ei_hbm.at[rw + j], eidx.at[e],
                                         ds_[e])

        def gather_cp(e, b):
            return pltpu.make_async_copy(hp_hbm.at[eidx.at[e, 0]],
                                         rows.at[b], gs[b])

        def scat_start(e, b):
            pltpu.async_copy(rows.at[b], acc_sh.at[eidx.at[e, 1]], ts[b],
                             add=True)

        def scat_wait(b):
            pltpu.make_async_copy(rows.at[b], acc_sh.at[eidx.at[0, 1]],
                                  ts[b]).wait()

        eidx_cp(0, 0).start()
        eidx_cp(1, 1).start()
        eidx_cp(0, 0).wait()
        gather_cp(0, 0).start()
        eidx_cp(1, 1).wait()
        gather_cp(1, 1).start()

        def body(jj, carry):
            for b in (0, 1, 2):
                j = jj * 3 + b
                b2 = (b + 2) % 3

                @pl.when(j < n_w)
                def _process():
                    @pl.when(j + 2 < n_w)
                    def _free_and_prefetch():
                        @pl.when(j >= 1)
                        def _w():
                            scat_wait(b2)
                        eidx_cp(j + 2, b2).start()

                    gather_cp(b, b).wait()
                    scat_start(b, b)

                    @pl.when(j + 2 < n_w)
                    def _gather_next():
                        eidx_cp(j + 2, b2).wait()
                        gather_cp(b2, b2).start()
            return carry

        lax.fori_loop(0, (n_w + 2) // 3, body, 0)
        for b in (0, 1, 2):
            scat_wait(b)

        if rem:
            @pl.when(w < rem)
            def _tail():
                pltpu.sync_copy(ei_hbm.at[nch8 + w], eidx.at[0])
                gather_cp(0, 0).start()
                gather_cp(0, 0).wait()
                scat_start(0, 0)
                scat_wait(0)

        plsc.subcore_barrier()

        def wb(i, phase):
            q0, qn = wb_chunks[i]
            b = i % 2
            cp = pltpu.make_async_copy(
                rows.at[b, pl.ds(0, qn)],
                out_hbm.at[pl.ds(c * NPAD + r0 + q0, qn), :], gs[b])
            if phase == 0:
                pltpu.sync_copy(acc_sh.at[pl.ds(r0 + q0, qn), :],
                                rows.at[b, pl.ds(0, qn)])
                cp.start()
            else:
                cp.wait()

        for i in range(len(wb_chunks)):
            if i >= 2:
                wb(i - 2, 1)
            wb(i, 0)
        for i in range(max(0, len(wb_chunks) - 2), len(wb_chunks)):
            wb(i, 1)

    zrows = jnp.zeros((CHD, D), jnp.float32)
    return agg_kernel(hp, ei3, zrows)


def _tc1_body(x_ref, w_ref, d0_ref, d1_ref, out_ref, dinv_ref):
    dinv = lax.rsqrt(d0_ref[...] + d1_ref[...] + 1.0)
    dinv_ref[...] = dinv
    h = jnp.dot(x_ref[...], w_ref[...], preferred_element_type=jnp.float32)
    out_ref[...] = h * dinv


def _tc1(x, w1, deg_col):
    return pl.pallas_call(
        _tc1_body,
        grid=(NBLK,),
        in_specs=[
            pl.BlockSpec((RPB, D), lambda i: (i, 0)),
            pl.BlockSpec((D, D), lambda i: (0, 0)),
            pl.BlockSpec((RPB, 1), lambda i: (i, 0)),
            pl.BlockSpec((RPB, 1), lambda i: (i + NBLK, 0)),
        ],
        out_specs=[
            pl.BlockSpec((RPB, D), lambda i: (i, 0)),
            pl.BlockSpec((RPB, 1), lambda i: (i, 0)),
        ],
        out_shape=[
            jax.ShapeDtypeStruct((NPAD, D), jnp.float32),
            jax.ShapeDtypeStruct((NPAD, 1), jnp.float32),
        ],
    )(x, w1, deg_col, deg_col)


def _tc2_body(a0_ref, a1_ref, hp_ref, dinv_ref, b_ref, w_ref, out_ref):
    pre = dinv_ref[...] * (a0_ref[...] + a1_ref[...] + hp_ref[...]) + b_ref[...]
    x2 = jnp.maximum(pre, 0.0)
    h = jnp.dot(x2, w_ref[...], preferred_element_type=jnp.float32)
    out_ref[...] = h * dinv_ref[...]


def _tc2(g1, h1p, dinv_col, b1v, w2):
    return pl.pallas_call(
        _tc2_body,
        grid=(NBLK,),
        in_specs=[
            pl.BlockSpec((RPB, D), lambda i: (i, 0)),
            pl.BlockSpec((RPB, D), lambda i: (i + NBLK, 0)),
            pl.BlockSpec((RPB, D), lambda i: (i, 0)),
            pl.BlockSpec((RPB, 1), lambda i: (i, 0)),
            pl.BlockSpec((D,), lambda i: (0,)),
            pl.BlockSpec((D, D), lambda i: (0, 0)),
        ],
        out_specs=pl.BlockSpec((RPB, D), lambda i: (i, 0)),
        out_shape=jax.ShapeDtypeStruct((NPAD, D), jnp.float32),
    )(g1, g1, h1p, dinv_col, b1v, w2)


def _tc3_body(a0_ref, a1_ref, hp_ref, dinv_ref, b_ref, out_ref):
    pre = dinv_ref[...] * (a0_ref[...] + a1_ref[...] + hp_ref[...]) + b_ref[...]
    out_ref[...] = jnp.maximum(pre, 0.0)


def _tc3(g2, h2p, dinv_col, b2v):
    return pl.pallas_call(
        _tc3_body,
        grid=(NBLK,),
        in_specs=[
            pl.BlockSpec((RPB, D), lambda i: (i, 0)),
            pl.BlockSpec((RPB, D), lambda i: (i + NBLK, 0)),
            pl.BlockSpec((RPB, D), lambda i: (i, 0)),
            pl.BlockSpec((RPB, 1), lambda i: (i, 0)),
            pl.BlockSpec((D,), lambda i: (0,)),
        ],
        out_specs=pl.BlockSpec((RPB, D), lambda i: (i, 0)),
        out_shape=jax.ShapeDtypeStruct((NNODES, D), jnp.float32),
    )(g2, g2, h2p, dinv_col, b2v)


def kernel(x, edge_index, W1, b1, W2, b2):
    e = edge_index.shape[1]
    tot = -(-e // CHD) * CHD
    pad = tot - e
    ei = edge_index
    if pad:
        ar = jnp.arange(pad, dtype=jnp.int32)
        dummy = jnp.stack([ar % NNODES, NNODES + ar % (NPAD - NNODES)])
        ei = jnp.concatenate([ei, dummy], axis=1)
    nch = tot // CHD
    nch8 = nch // 8 * 8
    rem = nch - nch8
    bounds = _chunk_bounds(nch8)
    max_n = max(y - x for x, y in zip(bounds[:-1], bounds[1:]))
    if bounds[-2] + max_n > nch:
        raise NotImplementedError("edge count too small for this layout")
    ei3 = ei.reshape(2, nch, CHD).transpose(1, 0, 2)

    deg2 = _deg_call(ei3, nch8, max_n, rem)
    deg_col = deg2.reshape(NC * NPAD, 1)

    h1p, dinv_col = _tc1(x, W1, deg_col)
    g1 = _agg_call(h1p, ei3, nch8, rem)
    h2p = _tc2(g1, h1p, dinv_col, b1, W2)
    g2 = _agg_call(h2p, ei3, nch8, rem)
    return _tc3(g2, h2p, dinv_col, b2)
